# Initial kernel scaffold; baseline (speedup 1.0000x reference)
import jax
import jax.numpy as jnp
from jax.experimental import pallas as pl


def kernel(y, r, mems, edge_index, W):
    raise NotImplementedError("write your pallas kernel here")


# speedup vs baseline: 6.6560x; 6.6560x over previous
import functools

import jax
import jax.numpy as jnp
from jax.experimental import pallas as pl
from jax.experimental.pallas import tpu as pltpu

MEM_LEN = 100000
Z_DIM = 128
K = 7

BLK = 1000


def _xw_body(y_ref, r_ref, mems_ref, w_ref, xw_ref, dist_ref):
    i = pl.program_id(0)
    y = y_ref[0]
    rows = mems_ref[...]
    ridx = jax.lax.broadcasted_iota(jnp.int32, (BLK, 1), 0) + i * BLK
    rvec = r_ref[...]
    rows = jnp.where(ridx == y, rvec, rows)
    xw_ref[...] = jnp.dot(rows, w_ref[...], preferred_element_type=jnp.float32)
    d = rows - rvec
    dist_ref[...] = jnp.sum(d * d, axis=1, keepdims=True)


def _compute_xw_dist(y, r, mems, W):
    grid = (MEM_LEN // BLK,)
    xw, dist = pl.pallas_call(
        _xw_body,
        grid=grid,
        in_specs=[
            pl.BlockSpec(memory_space=pltpu.SMEM),
            pl.BlockSpec((1, Z_DIM), lambda i: (0, 0)),
            pl.BlockSpec((BLK, Z_DIM), lambda i: (i, 0)),
            pl.BlockSpec((Z_DIM, Z_DIM), lambda i: (0, 0)),
        ],
        out_specs=[
            pl.BlockSpec((BLK, Z_DIM), lambda i: (i, 0)),
            pl.BlockSpec((BLK, 1), lambda i: (i, 0)),
        ],
        out_shape=[
            jax.ShapeDtypeStruct((MEM_LEN, Z_DIM), jnp.float32),
            jax.ShapeDtypeStruct((MEM_LEN, 1), jnp.float32),
        ],
    )(y.astype(jnp.int32), r, mems, W)
    return xw, dist[:, 0]


def kernel(y, r, mems, edge_index, W):
    xw, dist = _compute_xw_dist(y, r, mems, W)
    _, nn_idx = jax.lax.top_k(-dist, K)

    src = edge_index[0]
    dst = edge_index[1]
    y0 = y[0]

    ones = jnp.ones((src.shape[0],), jnp.float32)
    deg = jax.ops.segment_sum(ones, dst, num_segments=MEM_LEN) + 2.0
    deg = deg.at[y0].set(float(K + 2))
    dinv = jax.lax.rsqrt(deg)

    u = dinv[:, None] * xw
    s = jnp.zeros((MEM_LEN, Z_DIM), jnp.float32).at[dst].add(u[src])
    s = s.at[y0].set(jnp.sum(u[nn_idx], axis=0))
    out = dinv[:, None] * (s + 2.0 * u)
    return out


# speedup vs baseline: 21.5764x; 3.2416x over previous
import jax
import jax.numpy as jnp
from jax import lax
from jax.experimental import pallas as pl
from jax.experimental.pallas import tpu as pltpu
from jax.experimental.pallas import tpu_sc as plsc

M = 100000
Z = 128
KNN = 7
BLK = 1024
GRID = 98
MP = GRID * BLK
R = 8192
NCH = 14
SP = NCH * R
E_PAD = 503808
EPT = E_PAD // 16
EROWS_H = 4096
E_PAD_H = EROWS_H * 128
HROWS = EROWS_H // 32
HIST_PT = SP // 16
CAP = 5120
ACC_PT = 520
OUT_PT = R // 16
BIGF = 3e38
BIGI = 2 ** 30

_mesh_cache = []


def _get_mesh():
    if not _mesh_cache:
        _mesh_cache.append(plsc.VectorSubcoreMesh(
            core_axis_name="c", subcore_axis_name="s"))
    return _mesh_cache[0]


def _hist_body(dst_ref, h_ref, dst_v, ones_v, zbuf, h_sp):
    c = lax.axis_index("c")
    s = lax.axis_index("s")
    w = c * 16 + s

    def zb(i, _):
        zbuf[pl.ds(i * 16, 16)] = jnp.zeros((16,), jnp.float32)
        return 0

    lax.fori_loop(0, HIST_PT // 16, zb, 0)

    def ob(i, _):
        ones_v[pl.ds(i * 16, 16)] = jnp.ones((16,), jnp.float32)
        return 0

    lax.fori_loop(0, 8, ob, 0)

    pltpu.sync_copy(dst_ref.at[pl.ds(w * HROWS, HROWS)], dst_v)
    pltpu.sync_copy(zbuf, h_sp.at[pl.ds(s * HIST_PT, HIST_PT)])
    plsc.subcore_barrier()

    def add(j, _):
        pltpu.sync_copy(ones_v, h_sp.at[dst_v.at[j]], add=True)
        return 0

    lax.fori_loop(0, HROWS, add, 0)
    plsc.subcore_barrier()
    pltpu.sync_copy(h_sp.at[pl.ds(s * HIST_PT, HIST_PT)],
                    h_ref.at[c, pl.ds(s * HIST_PT, HIST_PT)])


def _hist_call(dst2d):
    f = pl.kernel(
        _hist_body,
        out_type=jax.ShapeDtypeStruct((2, SP), jnp.float32),
        mesh=_get_mesh(),
        compiler_params=pltpu.CompilerParams(needs_layout_passes=False),
        scratch_types=[
            pltpu.VMEM((HROWS, 128), jnp.int32),
            pltpu.VMEM((128,), jnp.float32),
            pltpu.VMEM((HIST_PT,), jnp.float32),
            pltpu.VMEM_SHARED((SP,), jnp.float32),
        ],
    )
    return f(dst2d)


def _xw_body(y_ref, r_ref, mems_ref, w_ref, xw_ref, dist_ref):
    i = pl.program_id(0)
    y = y_ref[0]
    rows = mems_ref[...]
    ridx = jax.lax.broadcasted_iota(jnp.int32, (BLK, 1), 0) + i * BLK
    rvec = r_ref[...]
    sel = jnp.broadcast_to(ridx == y, (BLK, Z))
    rows = jnp.where(sel, jnp.broadcast_to(rvec, (BLK, Z)), rows)
    xw_ref[...] = jnp.dot(rows, w_ref[...], preferred_element_type=jnp.float32)
    mr = lax.dot_general(rvec, rows, (((1,), (1,)), ((), ())),
                         preferred_element_type=jnp.float32)
    sq = rows * rows
    ssq = lax.dot_general(jnp.ones((1, Z), jnp.float32), sq,
                          (((1,), (1,)), ((), ())),
                          preferred_element_type=jnp.float32)
    rss = jnp.sum(rvec * rvec)
    dist = ssq - 2.0 * mr + rss
    gc = jax.lax.broadcasted_iota(jnp.int32, (1, BLK), 1) + i * BLK
    dist = jnp.where(gc < M, dist, BIGF)
    dist_ref[...] = dist.reshape(1, 1, BLK)


def _xw_call(y, r, mems, W):
    xw, dist3 = pl.pallas_call(
        _xw_body,
        grid=(GRID,),
        in_specs=[
            pl.BlockSpec(memory_space=pltpu.SMEM),
            pl.BlockSpec((1, Z), lambda i: (0, 0)),
            pl.BlockSpec((BLK, Z), lambda i: (i, 0)),
            pl.BlockSpec((Z, Z), lambda i: (0, 0)),
        ],
        out_specs=[
            pl.BlockSpec((BLK, Z), lambda i: (i, 0)),
            pl.BlockSpec((1, 1, BLK), lambda i: (i, 0, 0)),
        ],
        out_shape=[
            jax.ShapeDtypeStruct((M, Z), jnp.float32),
            jax.ShapeDtypeStruct((GRID, 1, BLK), jnp.float32),
        ],
    )(y.astype(jnp.int32), r, mems, W)
    return xw, dist3


def _topk_body(dist_ref, out_ref, xs_ref):
    xs_ref[...] = dist_ref[...].reshape(GRID, BLK)
    fi = (jax.lax.broadcasted_iota(jnp.int32, (GRID, BLK), 0) * BLK
          + jax.lax.broadcasted_iota(jnp.int32, (GRID, BLK), 1))
    for k in range(KNN):
        x = xs_ref[...]
        m = jnp.min(x)
        idx = jnp.min(jnp.where(x == m, fi, BIGI))
        out_ref[k] = idx
        xs_ref[...] = jnp.where(fi == idx, BIGF, x)
    out_ref[7] = 0


def _topk_call(dist3):
    nn = pl.pallas_call(
        _topk_body,
        in_specs=[pl.BlockSpec((GRID, 1, BLK), lambda: (0, 0, 0))],
        out_specs=pl.BlockSpec(memory_space=pltpu.SMEM),
        out_shape=jax.ShapeDtypeStruct((8,), jnp.int32),
        scratch_shapes=[pltpu.VMEM((GRID, BLK), jnp.float32)],
    )(dist3)
    return nn


NWIN = 8
WIN = EPT // NWIN
NVREG = WIN // 16
BCAP = 4096
SRCMASK = (1 << 17) - 1


def _seg_body(u_ref, src_ref, dst_ref, s_ref,
              wsrc, wdst, bucket, stage_src, stage_dst, rowbuf, zrow,
              acc, sem):
    c = lax.axis_index("c")
    s = lax.axis_index("s")
    lane = lax.iota(jnp.int32, 16)
    z16f = jnp.zeros((16,), jnp.float32)
    cbase = c * (NCH // 2)

    def zb(i, _):
        zrow[i >> 3, pl.ds((i & 7) * 16, 16)] = z16f
        return 0

    lax.fori_loop(0, 40 * 8, zb, 0)

    cnts = tuple(jnp.zeros((16,), jnp.int32) for _ in range(NCH // 2))
    for wi in range(NWIN):
        off = s * EPT + wi * WIN
        pltpu.sync_copy(src_ref.at[pl.ds(off, WIN)], wsrc)
        pltpu.sync_copy(dst_ref.at[pl.ds(off, WIN)], wdst)

        def win_body(j, cnts):
            d = wdst[pl.ds(j * 16, 16)]
            sv = wsrc[pl.ds(j * 16, 16)]
            chrel = (d >> 13) - cbase
            val = sv | ((d & (R - 1)) << 17)
            new = []
            for pb in range(NCH // 2):
                m = chrel == pb
                pos = cnts[pb] + plsc.cumsum(m.astype(jnp.int32)) - 1
                plsc.store_scatter(bucket, [pb * BCAP + pos], val, mask=m)
                new.append(cnts[pb] + plsc.all_reduce_population_count(m))
            return tuple(new)

        cnts = lax.fori_loop(0, NVREG, win_body, cnts)

    for pb in range(NCH // 2):
        base = (cbase + pb) * R
        for z in range(13):
            pltpu.sync_copy(zrow, acc.at[pl.ds(s * ACC_PT + z * 40, 40)])
        plsc.subcore_barrier()

        cnt = cnts[pb]
        cnt_s = jnp.max(cnt)
        ncap_s = (cnt_s + 127) & (-128)
        for tt in range(8):
            posv = cnt + tt * 16 + lane
            mm = posv < ncap_s
            padval = (R + (posv & 15)) << 17
            plsc.store_scatter(bucket, [pb * BCAP + posv], padval, mask=mm)

        def gath(g, _):
            for tt in range(8):
                v = bucket[pl.ds(pb * BCAP + g * 128 + tt * 16, 16)]
                stage_src[pl.ds(tt * 16, 16)] = v & SRCMASK
                stage_dst[0, pl.ds(tt * 16, 16)] = v >> 17
            pltpu.async_copy(u_ref.at[stage_src], rowbuf, sem).wait()
            pltpu.sync_copy(rowbuf, acc.at[stage_dst.at[0]], add=True)
            return 0

        lax.fori_loop(0, ncap_s >> 7, gath, 0)
        plsc.subcore_barrier()
        pltpu.sync_copy(acc.at[pl.ds(s * OUT_PT, OUT_PT)],
                        s_ref.at[pl.ds(base + s * OUT_PT, OUT_PT)])
        plsc.subcore_barrier()


def _seg_call(u, src1d, dst1d):
    f = pl.kernel(
        _seg_body,
        out_type=jax.ShapeDtypeStruct((SP, Z), jnp.float32),
        mesh=_get_mesh(),
        compiler_params=pltpu.CompilerParams(needs_layout_passes=False),
        scratch_types=[
            pltpu.VMEM((WIN,), jnp.int32),
            pltpu.VMEM((WIN,), jnp.int32),
            pltpu.VMEM((7 * BCAP,), jnp.int32),
            pltpu.VMEM((128,), jnp.int32),
            pltpu.VMEM((1, 128), jnp.int32),
            pltpu.VMEM((128, Z), jnp.float32),
            pltpu.VMEM((40, 128), jnp.float32),
            pltpu.VMEM_SHARED((ACC_PT * 16, Z), jnp.float32),
            pltpu.SemaphoreType.DMA,
        ],
    )
    return f(u, src1d, dst1d)


def kernel(y, r, mems, edge_index, W):
    y0 = y[0].astype(jnp.int32)
    E = edge_index.shape[1]
    src_e = edge_index[0].astype(jnp.int32)
    dst_e = edge_index[1].astype(jnp.int32)
    pad_n = E_PAD - E - KNN

    spread_e = MP + (jnp.arange(E, dtype=jnp.int32) % (SP - MP))
    dst_m = jnp.where(dst_e == y0, spread_e, dst_e)
    pad_dst = MP + (jnp.arange(pad_n, dtype=jnp.int32) % (SP - MP))
    pad_src = jnp.arange(pad_n, dtype=jnp.int32) % M
    pad_n_h = E_PAD_H - E - KNN
    pad_dst_h = MP + (jnp.arange(pad_n_h, dtype=jnp.int32) % (SP - MP))

    dst_hist = jnp.concatenate(
        [dst_m, jnp.full((KNN,), y0, jnp.int32), pad_dst_h])
    h = _hist_call(dst_hist.reshape(EROWS_H, 128))

    deg = h[0, :M] + h[1, :M] + 2.0
    deg = deg.at[y0].set(float(KNN + 2))
    dinv = jax.lax.rsqrt(deg)

    xw, dist3 = _xw_call(y, r, mems, W)
    nn8 = _topk_call(dist3)
    nn_idx = nn8[:KNN]

    u = dinv[:, None] * xw
    src_all = jnp.concatenate([src_e, nn_idx, pad_src])
    dst_all = jnp.concatenate(
        [dst_m, jnp.full((KNN,), y0, jnp.int32), pad_dst])
    s = _seg_call(u, src_all, dst_all)

    out = dinv[:, None] * s[:M] + (2.0 * dinv * dinv)[:, None] * xw
    return out


# speedup vs baseline: 36.5526x; 1.6941x over previous
import jax
import jax.numpy as jnp
from jax import lax
from jax.experimental import pallas as pl
from jax.experimental.pallas import tpu as pltpu
from jax.experimental.pallas import tpu_sc as plsc

M = 100000
Z = 128
KNN = 7
BLK = 1024
GRID = 98
MP = GRID * BLK
R = 8192
NCH = 14
SP = NCH * R
E_PAD = 503808
EPT = E_PAD // 16
EROWS_H = 4096
E_PAD_H = EROWS_H * 128
HROWS = EROWS_H // 32
HIST_PT = SP // 16
CAP = 5120
ACC_PT = 520
OUT_PT = R // 16
BIGF = 3e38
BIGI = 2 ** 30

_mesh_cache = []


def _get_mesh():
    if not _mesh_cache:
        _mesh_cache.append(plsc.VectorSubcoreMesh(
            core_axis_name="c", subcore_axis_name="s"))
    return _mesh_cache[0]


def _hist_body(dst_ref, h_ref, dst_v, ones_v, zbuf, h_sp):
    c = lax.axis_index("c")
    s = lax.axis_index("s")
    w = c * 16 + s

    def zb(i, _):
        zbuf[pl.ds(i * 16, 16)] = jnp.zeros((16,), jnp.float32)
        return 0

    lax.fori_loop(0, HIST_PT // 16, zb, 0)

    def ob(i, _):
        ones_v[pl.ds(i * 16, 16)] = jnp.ones((16,), jnp.float32)
        return 0

    lax.fori_loop(0, 8, ob, 0)

    pltpu.sync_copy(dst_ref.at[pl.ds(w * HROWS, HROWS)], dst_v)
    pltpu.sync_copy(zbuf, h_sp.at[pl.ds(s * HIST_PT, HIST_PT)])
    plsc.subcore_barrier()

    def add(j, _):
        pltpu.sync_copy(ones_v, h_sp.at[dst_v.at[j]], add=True)
        return 0

    lax.fori_loop(0, HROWS, add, 0)
    plsc.subcore_barrier()
    pltpu.sync_copy(h_sp.at[pl.ds(s * HIST_PT, HIST_PT)],
                    h_ref.at[c, pl.ds(s * HIST_PT, HIST_PT)])


def _hist_call(dst2d):
    f = pl.kernel(
        _hist_body,
        out_type=jax.ShapeDtypeStruct((2, SP), jnp.float32),
        mesh=_get_mesh(),
        compiler_params=pltpu.CompilerParams(needs_layout_passes=False),
        scratch_types=[
            pltpu.VMEM((HROWS, 128), jnp.int32),
            pltpu.VMEM((128,), jnp.float32),
            pltpu.VMEM((HIST_PT,), jnp.float32),
            pltpu.VMEM_SHARED((SP,), jnp.float32),
        ],
    )
    return f(dst2d)


def _xw_body(y_ref, r_ref, mems_ref, w_ref, xw_ref, dist_ref):
    i = pl.program_id(0)
    y = y_ref[0]
    rows = mems_ref[...]
    ridx = jax.lax.broadcasted_iota(jnp.int32, (BLK, 1), 0) + i * BLK
    rvec = r_ref[...]
    sel = jnp.broadcast_to(ridx == y, (BLK, Z))
    rows = jnp.where(sel, jnp.broadcast_to(rvec, (BLK, Z)), rows)
    rows_ok = jnp.broadcast_to(ridx < M, (BLK, Z))
    xw_ref[...] = jnp.where(
        rows_ok,
        jnp.dot(rows, w_ref[...], preferred_element_type=jnp.float32), 0.0)
    mr = lax.dot_general(rvec, rows, (((1,), (1,)), ((), ())),
                         preferred_element_type=jnp.float32)
    sq = rows * rows
    ssq = lax.dot_general(jnp.ones((1, Z), jnp.float32), sq,
                          (((1,), (1,)), ((), ())),
                          preferred_element_type=jnp.float32)
    rss = jnp.sum(rvec * rvec)
    dist = ssq - 2.0 * mr + rss
    gc = jax.lax.broadcasted_iota(jnp.int32, (1, BLK), 1) + i * BLK
    dist = jnp.where(gc < M, dist, BIGF)
    dist_ref[...] = dist.reshape(1, 1, BLK)


def _xw_call(y, r, mems, W):
    xw, dist3 = pl.pallas_call(
        _xw_body,
        grid=(GRID,),
        in_specs=[
            pl.BlockSpec(memory_space=pltpu.SMEM),
            pl.BlockSpec((1, Z), lambda i: (0, 0)),
            pl.BlockSpec((BLK, Z), lambda i: (i, 0)),
            pl.BlockSpec((Z, Z), lambda i: (0, 0)),
        ],
        out_specs=[
            pl.BlockSpec((BLK, Z), lambda i: (i, 0)),
            pl.BlockSpec((1, 1, BLK), lambda i: (i, 0, 0)),
        ],
        out_shape=[
            jax.ShapeDtypeStruct((MP, Z), jnp.float32),
            jax.ShapeDtypeStruct((GRID, 1, BLK), jnp.float32),
        ],
    )(y.astype(jnp.int32), r, mems, W)
    return xw, dist3


def _topk_body(dist_ref, out_ref, xs_ref):
    xs_ref[...] = dist_ref[...].reshape(GRID, BLK)
    fi = (jax.lax.broadcasted_iota(jnp.int32, (GRID, BLK), 0) * BLK
          + jax.lax.broadcasted_iota(jnp.int32, (GRID, BLK), 1))
    for k in range(KNN):
        x = xs_ref[...]
        m = jnp.min(x)
        idx = jnp.min(jnp.where(x == m, fi, BIGI))
        out_ref[k] = idx
        xs_ref[...] = jnp.where(fi == idx, BIGF, x)
    out_ref[7] = 0


def _topk_call(dist3):
    nn = pl.pallas_call(
        _topk_body,
        in_specs=[pl.BlockSpec((GRID, 1, BLK), lambda: (0, 0, 0))],
        out_specs=pl.BlockSpec(memory_space=pltpu.SMEM),
        out_shape=jax.ShapeDtypeStruct((8,), jnp.int32),
        scratch_shapes=[pltpu.VMEM((GRID, BLK), jnp.float32)],
    )(dist3)
    return nn


NWIN = 16
WIN = EPT // NWIN
NVREG = WIN // 16
BCAP = 3072
SRCMASK = (1 << 17) - 1


def _seg_body(u_ref, src_ref, dst_ref, s_ref,
              wsrc, wdst, bucket, ss0, sd0, ss1, sd1, rowbuf0, rowbuf1,
              zrow, acc, semz, semg0, semg1, sema0, sema1):
    c = lax.axis_index("c")
    s = lax.axis_index("s")
    lane = lax.iota(jnp.int32, 16)
    z16f = jnp.zeros((16,), jnp.float32)
    cbase = c * (NCH // 2)

    def zb(i, _):
        zrow[i >> 3, pl.ds((i & 7) * 16, 16)] = z16f
        return 0

    lax.fori_loop(0, 32 * 8, zb, 0)

    cnts = tuple(jnp.zeros((16,), jnp.int32) for _ in range(NCH // 2))
    for wi in range(NWIN):
        off = s * EPT + wi * WIN
        pltpu.sync_copy(src_ref.at[pl.ds(off, WIN)], wsrc)
        pltpu.sync_copy(dst_ref.at[pl.ds(off, WIN)], wdst)

        def win_body(j, cnts):
            d = wdst[pl.ds(j * 16, 16)]
            sv = wsrc[pl.ds(j * 16, 16)]
            chrel = (d >> 13) - cbase
            val = sv | ((d & (R - 1)) << 17)
            new = []
            for pb in range(NCH // 2):
                m = chrel == pb
                pos = cnts[pb] + plsc.cumsum(m.astype(jnp.int32)) - 1
                plsc.store_scatter(bucket, [pb * BCAP + pos], val, mask=m)
                new.append(cnts[pb] + plsc.all_reduce_population_count(m))
            return tuple(new)

        cnts = lax.fori_loop(0, NVREG, win_body, cnts)

    for pb in range(NCH // 2):
        base = (cbase + pb) * R
        zd = [pltpu.async_copy(zrow, acc.at[pl.ds(s * OUT_PT + z * 32, 32)],
                               semz) for z in range(16)]
        for d in zd:
            d.wait()
        plsc.subcore_barrier()

        cnt = cnts[pb]
        cnt_s = jnp.max(cnt)
        ncap_s = (cnt_s + 255) & (-256)
        for tt in range(16):
            posv = cnt + tt * 16 + lane
            mm = posv < ncap_s
            padval = M + (posv & 255)
            plsc.store_scatter(bucket, [pb * BCAP + posv], padval, mask=mm)

        def unpack(goff, stage_s, stage_d):
            for tt in range(8):
                v = bucket[pl.ds(pb * BCAP + goff * 128 + tt * 16, 16)]
                stage_s[pl.ds(tt * 16, 16)] = v & SRCMASK
                stage_d[0, pl.ds(tt * 16, 16)] = v >> 17

        def pair(i, _):
            g0 = i * 2
            unpack(g0, ss0, sd0)
            d0 = pltpu.async_copy(u_ref.at[ss0], rowbuf0, semg0)
            unpack(g0 + 1, ss1, sd1)
            d1 = pltpu.async_copy(u_ref.at[ss1], rowbuf1, semg1)
            d0.wait()
            a0 = pltpu.async_copy(rowbuf0, acc.at[sd0.at[0]], sema0, add=True)
            d1.wait()
            a1 = pltpu.async_copy(rowbuf1, acc.at[sd1.at[0]], sema1, add=True)
            a0.wait()
            a1.wait()
            return 0

        lax.fori_loop(0, ncap_s >> 8, pair, 0)
        plsc.subcore_barrier()
        pltpu.sync_copy(acc.at[pl.ds(s * OUT_PT, OUT_PT)],
                        s_ref.at[pl.ds(base + s * OUT_PT, OUT_PT)])
        plsc.subcore_barrier()


def _seg_call(u, src1d, dst1d):
    f = pl.kernel(
        _seg_body,
        out_type=jax.ShapeDtypeStruct((SP, Z), jnp.float32),
        mesh=_get_mesh(),
        compiler_params=pltpu.CompilerParams(needs_layout_passes=False),
        scratch_types=[
            pltpu.VMEM((WIN,), jnp.int32),
            pltpu.VMEM((WIN,), jnp.int32),
            pltpu.VMEM((7 * BCAP,), jnp.int32),
            pltpu.VMEM((128,), jnp.int32),
            pltpu.VMEM((1, 128), jnp.int32),
            pltpu.VMEM((128,), jnp.int32),
            pltpu.VMEM((1, 128), jnp.int32),
            pltpu.VMEM((128, Z), jnp.float32),
            pltpu.VMEM((128, Z), jnp.float32),
            pltpu.VMEM((32, 128), jnp.float32),
            pltpu.VMEM_SHARED((R, Z), jnp.float32),
            pltpu.SemaphoreType.DMA,
            pltpu.SemaphoreType.DMA,
            pltpu.SemaphoreType.DMA,
            pltpu.SemaphoreType.DMA,
            pltpu.SemaphoreType.DMA,
        ],
    )
    return f(u, src1d, dst1d)


def kernel(y, r, mems, edge_index, W):
    y0 = y[0].astype(jnp.int32)
    E = edge_index.shape[1]
    src_e = edge_index[0].astype(jnp.int32)
    dst_e = edge_index[1].astype(jnp.int32)
    pad_n = E_PAD - E - KNN

    spread_e = MP + (jnp.arange(E, dtype=jnp.int32) % (SP - MP))
    dst_m = jnp.where(dst_e == y0, spread_e, dst_e)
    pad_dst = MP + (jnp.arange(pad_n, dtype=jnp.int32) % (SP - MP))
    pad_src = jnp.arange(pad_n, dtype=jnp.int32) % M
    pad_n_h = E_PAD_H - E - KNN
    pad_dst_h = MP + (jnp.arange(pad_n_h, dtype=jnp.int32) % (SP - MP))

    dst_hist = jnp.concatenate(
        [dst_m, jnp.full((KNN,), y0, jnp.int32), pad_dst_h])
    h = _hist_call(dst_hist.reshape(EROWS_H, 128))

    deg = h[0, :MP] + h[1, :MP] + 2.0
    deg = deg.at[y0].set(float(KNN + 2))
    dinv = jax.lax.rsqrt(deg)

    xw, dist3 = _xw_call(y, r, mems, W)
    nn8 = _topk_call(dist3)
    nn_idx = nn8[:KNN]

    u = dinv[:, None] * xw
    src_all = jnp.concatenate([src_e, nn_idx, pad_src])
    dst_all = jnp.concatenate(
        [dst_m, jnp.full((KNN,), y0, jnp.int32), pad_dst])
    s = _seg_call(u, src_all, dst_all)

    dv = dinv[:M]
    out = dv[:, None] * s[:M] + (2.0 * dv * dv)[:, None] * xw[:M]
    return out


# speedup vs baseline: 37.3875x; 1.0228x over previous
import jax
import jax.numpy as jnp
from jax import lax
from jax.experimental import pallas as pl
from jax.experimental.pallas import tpu as pltpu
from jax.experimental.pallas import tpu_sc as plsc

M = 100000
Z = 128
KNN = 7
BLK = 1024
GRID = 98
MP = GRID * BLK
R = 8192
NCH = 14
SP = NCH * R
E_PAD = 503808
EPT = E_PAD // 16
EROWS_H = 4096
E_PAD_H = EROWS_H * 128
HROWS = EROWS_H // 32
HIST_PT = SP // 16
CAP = 5120
ACC_PT = 520
OUT_PT = R // 16
BIGF = 3e38
BIGI = 2 ** 30

_mesh_cache = []


def _get_mesh():
    if not _mesh_cache:
        _mesh_cache.append(plsc.VectorSubcoreMesh(
            core_axis_name="c", subcore_axis_name="s"))
    return _mesh_cache[0]


def _hist_body(dst_ref, h_ref, dst_v, ones_v, zbuf, h_sp):
    c = lax.axis_index("c")
    s = lax.axis_index("s")
    w = c * 16 + s

    def zb(i, _):
        zbuf[pl.ds(i * 16, 16)] = jnp.zeros((16,), jnp.float32)
        return 0

    lax.fori_loop(0, HIST_PT // 16, zb, 0)

    def ob(i, _):
        ones_v[pl.ds(i * 16, 16)] = jnp.ones((16,), jnp.float32)
        return 0

    lax.fori_loop(0, 8, ob, 0)

    pltpu.sync_copy(dst_ref.at[pl.ds(w * HROWS, HROWS)], dst_v)
    pltpu.sync_copy(zbuf, h_sp.at[pl.ds(s * HIST_PT, HIST_PT)])
    plsc.subcore_barrier()

    def add(j, _):
        pltpu.sync_copy(ones_v, h_sp.at[dst_v.at[j]], add=True)
        return 0

    lax.fori_loop(0, HROWS, add, 0)
    plsc.subcore_barrier()
    pltpu.sync_copy(h_sp.at[pl.ds(s * HIST_PT, HIST_PT)],
                    h_ref.at[c, pl.ds(s * HIST_PT, HIST_PT)])


def _hist_call(dst2d):
    f = pl.kernel(
        _hist_body,
        out_type=jax.ShapeDtypeStruct((2, SP), jnp.float32),
        mesh=_get_mesh(),
        compiler_params=pltpu.CompilerParams(needs_layout_passes=False),
        scratch_types=[
            pltpu.VMEM((HROWS, 128), jnp.int32),
            pltpu.VMEM((128,), jnp.float32),
            pltpu.VMEM((HIST_PT,), jnp.float32),
            pltpu.VMEM_SHARED((SP,), jnp.float32),
        ],
    )
    return f(dst2d)


def _xw_body(y_ref, r_ref, mems_ref, w_ref, xw_ref, dist_ref):
    i = pl.program_id(0)
    y = y_ref[0]
    rows = mems_ref[...]
    ridx = jax.lax.broadcasted_iota(jnp.int32, (BLK, 1), 0) + i * BLK
    rvec = r_ref[...]
    sel = jnp.broadcast_to(ridx == y, (BLK, Z))
    rows = jnp.where(sel, jnp.broadcast_to(rvec, (BLK, Z)), rows)
    rows_ok = jnp.broadcast_to(ridx < M, (BLK, Z))
    xw_ref[...] = jnp.where(
        rows_ok,
        jnp.dot(rows, w_ref[...], preferred_element_type=jnp.float32), 0.0)
    mr = lax.dot_general(rvec, rows, (((1,), (1,)), ((), ())),
                         preferred_element_type=jnp.float32)
    sq = rows * rows
    ssq = lax.dot_general(jnp.ones((1, Z), jnp.float32), sq,
                          (((1,), (1,)), ((), ())),
                          preferred_element_type=jnp.float32)
    rss = jnp.sum(rvec * rvec)
    dist = ssq - 2.0 * mr + rss
    gc = jax.lax.broadcasted_iota(jnp.int32, (1, BLK), 1) + i * BLK
    dist = jnp.where(gc < M, dist, BIGF)
    dist_ref[...] = dist.reshape(1, 1, BLK)


def _xw_call(y, r, mems, W):
    xw, dist3 = pl.pallas_call(
        _xw_body,
        grid=(GRID,),
        in_specs=[
            pl.BlockSpec(memory_space=pltpu.SMEM),
            pl.BlockSpec((1, Z), lambda i: (0, 0)),
            pl.BlockSpec((BLK, Z), lambda i: (i, 0)),
            pl.BlockSpec((Z, Z), lambda i: (0, 0)),
        ],
        out_specs=[
            pl.BlockSpec((BLK, Z), lambda i: (i, 0)),
            pl.BlockSpec((1, 1, BLK), lambda i: (i, 0, 0)),
        ],
        out_shape=[
            jax.ShapeDtypeStruct((MP, Z), jnp.float32),
            jax.ShapeDtypeStruct((GRID, 1, BLK), jnp.float32),
        ],
    )(y.astype(jnp.int32), r, mems, W)
    return xw, dist3


def _topk_body(dist_ref, out_ref, xs_ref):
    xs_ref[...] = dist_ref[...].reshape(GRID, BLK)
    fi = (jax.lax.broadcasted_iota(jnp.int32, (GRID, BLK), 0) * BLK
          + jax.lax.broadcasted_iota(jnp.int32, (GRID, BLK), 1))
    for k in range(KNN):
        x = xs_ref[...]
        m = jnp.min(x)
        idx = jnp.min(jnp.where(x == m, fi, BIGI))
        out_ref[k] = idx
        xs_ref[...] = jnp.where(fi == idx, BIGF, x)
    out_ref[7] = 0


def _topk_call(dist3):
    nn = pl.pallas_call(
        _topk_body,
        in_specs=[pl.BlockSpec((GRID, 1, BLK), lambda: (0, 0, 0))],
        out_specs=pl.BlockSpec(memory_space=pltpu.SMEM),
        out_shape=jax.ShapeDtypeStruct((8,), jnp.int32),
        scratch_shapes=[pltpu.VMEM((GRID, BLK), jnp.float32)],
    )(dist3)
    return nn


NWIN = 16
WIN = EPT // NWIN
NVREG = WIN // 16
BCAP = 3072
SRCMASK = (1 << 17) - 1


def _seg_body(u_ref, src_ref, dst_ref, s_ref,
              wsrc, wdst, bucket,
              ss0, sd0, ss1, sd1, ss2, sd2, ss3, sd3,
              rowbuf0, rowbuf1, zrow, acc,
              semz, semg0, semg1, sema0, sema1):
    c = lax.axis_index("c")
    s = lax.axis_index("s")
    lane = lax.iota(jnp.int32, 16)
    z16f = jnp.zeros((16,), jnp.float32)
    cbase = c * (NCH // 2)

    def zb(i, _):
        zrow[i >> 3, pl.ds((i & 7) * 16, 16)] = z16f
        return 0

    lax.fori_loop(0, 32 * 8, zb, 0)

    cnts = tuple(jnp.zeros((16,), jnp.int32) for _ in range(NCH // 2))
    for wi in range(NWIN):
        off = s * EPT + wi * WIN
        pltpu.sync_copy(src_ref.at[pl.ds(off, WIN)], wsrc)
        pltpu.sync_copy(dst_ref.at[pl.ds(off, WIN)], wdst)

        def win_body(j, cnts):
            d = wdst[pl.ds(j * 16, 16)]
            sv = wsrc[pl.ds(j * 16, 16)]
            chrel = (d >> 13) - cbase
            val = sv | ((d & (R - 1)) << 17)
            new = []
            for pb in range(NCH // 2):
                m = chrel == pb
                pos = cnts[pb] + plsc.cumsum(m.astype(jnp.int32)) - 1
                plsc.store_scatter(bucket, [pb * BCAP + pos], val, mask=m)
                new.append(cnts[pb] + plsc.all_reduce_population_count(m))
            return tuple(new)

        cnts = lax.fori_loop(0, NVREG, win_body, cnts)

    def unpack(goff, pb, stage_s, stage_d):
        for tt in range(8):
            v = bucket[pl.ds(pb * BCAP + goff * 128 + tt * 16, 16)]
            stage_s[pl.ds(tt * 16, 16)] = v & SRCMASK
            stage_d[0, pl.ds(tt * 16, 16)] = v >> 17

    for pb in range(NCH // 2):
        base = (cbase + pb) * R
        zd = [pltpu.async_copy(zrow, acc.at[pl.ds(s * OUT_PT + z * 32, 32)],
                               semz) for z in range(16)]

        cnt = cnts[pb]
        cnt_s = jnp.max(cnt)
        ncap_s = (cnt_s + 255) & (-256)
        for tt in range(16):
            posv = cnt + tt * 16 + lane
            mm = posv < ncap_s
            padval = M + (posv & 255)
            plsc.store_scatter(bucket, [pb * BCAP + posv], padval, mask=mm)

        for d in zd:
            d.wait()
        plsc.subcore_barrier()

        ngrp = ncap_s >> 7
        nq = ngrp >> 2

        def quad(i, _):
            g = i * 4
            unpack(g, pb, ss0, sd0)
            dA = pltpu.async_copy(u_ref.at[ss0], rowbuf0, semg0)
            unpack(g + 1, pb, ss1, sd1)
            dB = pltpu.async_copy(u_ref.at[ss1], rowbuf1, semg1)
            dA.wait()
            aA = pltpu.async_copy(rowbuf0, acc.at[sd0.at[0]], sema0,
                                  add=True)
            dB.wait()
            aB = pltpu.async_copy(rowbuf1, acc.at[sd1.at[0]], sema1,
                                  add=True)
            unpack(g + 2, pb, ss2, sd2)
            aA.wait()
            dC = pltpu.async_copy(u_ref.at[ss2], rowbuf0, semg0)
            unpack(g + 3, pb, ss3, sd3)
            aB.wait()
            dD = pltpu.async_copy(u_ref.at[ss3], rowbuf1, semg1)
            dC.wait()
            aC = pltpu.async_copy(rowbuf0, acc.at[sd2.at[0]], sema0,
                                  add=True)
            dD.wait()
            aD = pltpu.async_copy(rowbuf1, acc.at[sd3.at[0]], sema1,
                                  add=True)
            aC.wait()
            aD.wait()
            return 0

        lax.fori_loop(0, nq, quad, 0)

        @pl.when((ngrp & 3) == 2)
        def _():
            g = nq * 4
            unpack(g, pb, ss0, sd0)
            dA = pltpu.async_copy(u_ref.at[ss0], rowbuf0, semg0)
            unpack(g + 1, pb, ss1, sd1)
            dB = pltpu.async_copy(u_ref.at[ss1], rowbuf1, semg1)
            dA.wait()
            aA = pltpu.async_copy(rowbuf0, acc.at[sd0.at[0]], sema0,
                                  add=True)
            dB.wait()
            aB = pltpu.async_copy(rowbuf1, acc.at[sd1.at[0]], sema1,
                                  add=True)
            aA.wait()
            aB.wait()

        plsc.subcore_barrier()
        pltpu.sync_copy(acc.at[pl.ds(s * OUT_PT, OUT_PT)],
                        s_ref.at[pl.ds(base + s * OUT_PT, OUT_PT)])
        plsc.subcore_barrier()


def _seg_call(u, src1d, dst1d):
    f = pl.kernel(
        _seg_body,
        out_type=jax.ShapeDtypeStruct((SP, Z), jnp.float32),
        mesh=_get_mesh(),
        compiler_params=pltpu.CompilerParams(needs_layout_passes=False),
        scratch_types=[
            pltpu.VMEM((WIN,), jnp.int32),
            pltpu.VMEM((WIN,), jnp.int32),
            pltpu.VMEM((7 * BCAP,), jnp.int32),
            pltpu.VMEM((128,), jnp.int32),
            pltpu.VMEM((1, 128), jnp.int32),
            pltpu.VMEM((128,), jnp.int32),
            pltpu.VMEM((1, 128), jnp.int32),
            pltpu.VMEM((128,), jnp.int32),
            pltpu.VMEM((1, 128), jnp.int32),
            pltpu.VMEM((128,), jnp.int32),
            pltpu.VMEM((1, 128), jnp.int32),
            pltpu.VMEM((128, Z), jnp.float32),
            pltpu.VMEM((128, Z), jnp.float32),
            pltpu.VMEM((32, 128), jnp.float32),
            pltpu.VMEM_SHARED((R, Z), jnp.float32),
            pltpu.SemaphoreType.DMA,
            pltpu.SemaphoreType.DMA,
            pltpu.SemaphoreType.DMA,
            pltpu.SemaphoreType.DMA,
            pltpu.SemaphoreType.DMA,
        ],
    )
    return f(u, src1d, dst1d)


def kernel(y, r, mems, edge_index, W):
    y0 = y[0].astype(jnp.int32)
    E = edge_index.shape[1]
    src_e = edge_index[0].astype(jnp.int32)
    dst_e = edge_index[1].astype(jnp.int32)
    pad_n = E_PAD - E - KNN

    spread_e = MP + (jnp.arange(E, dtype=jnp.int32) % (SP - MP))
    dst_m = jnp.where(dst_e == y0, spread_e, dst_e)
    pad_dst = MP + (jnp.arange(pad_n, dtype=jnp.int32) % (SP - MP))
    pad_src = M + (jnp.arange(pad_n, dtype=jnp.int32) % (MP - M))

    pad_n_h = E_PAD_H - E
    pad_dst_h = MP + (jnp.arange(pad_n_h, dtype=jnp.int32) % (SP - MP))
    dst_hist = jnp.concatenate([dst_m, pad_dst_h])
    h = _hist_call(dst_hist.reshape(EROWS_H, 128))

    deg = h[0, :MP] + h[1, :MP] + 2.0
    deg = deg.at[y0].set(float(KNN + 2))
    dinv = jax.lax.rsqrt(deg)

    xw, dist3 = _xw_call(y, r, mems, W)
    nn8 = _topk_call(dist3)
    nn_idx = nn8[:KNN]

    u = dinv[:, None] * xw
    src_all = jnp.concatenate([src_e, nn_idx, pad_src])
    dst_all = jnp.concatenate(
        [dst_m, jnp.full((KNN,), y0, jnp.int32), pad_dst])
    s = _seg_call(u, src_all, dst_all)

    dv = dinv[:M]
    out = dv[:, None] * s[:M] + (2.0 * dv * dv)[:, None] * xw[:M]
    return out


# speedup vs baseline: 40.2972x; 1.0778x over previous
import jax
import jax.numpy as jnp
from jax import lax
from jax.experimental import pallas as pl
from jax.experimental.pallas import tpu as pltpu
from jax.experimental.pallas import tpu_sc as plsc

M = 100000
Z = 128
KNN = 7
BLK = 1024
GRID = 98
MP = GRID * BLK
R = 8192
NCH = 14
SP = NCH * R
E_PAD = 503808
EPT = E_PAD // 16
EROWS_H = 4096
E_PAD_H = EROWS_H * 128
HROWS = EROWS_H // 32
HIST_PT = SP // 16
CAP = 5120
ACC_PT = 520
OUT_PT = R // 16
BIGF = 3e38
BIGI = 2 ** 30

_mesh_cache = []


def _get_mesh():
    if not _mesh_cache:
        _mesh_cache.append(plsc.VectorSubcoreMesh(
            core_axis_name="c", subcore_axis_name="s"))
    return _mesh_cache[0]


def _hist_body(dst_ref, h_ref, dst_v, ones_v, zbuf, h_sp):
    c = lax.axis_index("c")
    s = lax.axis_index("s")
    w = c * 16 + s

    def zb(i, _):
        zbuf[pl.ds(i * 16, 16)] = jnp.zeros((16,), jnp.float32)
        return 0

    lax.fori_loop(0, HIST_PT // 16, zb, 0)

    def ob(i, _):
        ones_v[pl.ds(i * 16, 16)] = jnp.ones((16,), jnp.float32)
        return 0

    lax.fori_loop(0, 8, ob, 0)

    pltpu.sync_copy(dst_ref.at[pl.ds(w * HROWS, HROWS)], dst_v)
    pltpu.sync_copy(zbuf, h_sp.at[pl.ds(s * HIST_PT, HIST_PT)])
    plsc.subcore_barrier()

    def add(j, _):
        pltpu.sync_copy(ones_v, h_sp.at[dst_v.at[j]], add=True)
        return 0

    lax.fori_loop(0, HROWS, add, 0)
    plsc.subcore_barrier()
    pltpu.sync_copy(h_sp.at[pl.ds(s * HIST_PT, HIST_PT)],
                    h_ref.at[c, pl.ds(s * HIST_PT, HIST_PT)])


def _hist_call(dst2d):
    f = pl.kernel(
        _hist_body,
        out_type=jax.ShapeDtypeStruct((2, SP), jnp.float32),
        mesh=_get_mesh(),
        compiler_params=pltpu.CompilerParams(needs_layout_passes=False),
        scratch_types=[
            pltpu.VMEM((HROWS, 128), jnp.int32),
            pltpu.VMEM((128,), jnp.float32),
            pltpu.VMEM((HIST_PT,), jnp.float32),
            pltpu.VMEM_SHARED((SP,), jnp.float32),
        ],
    )
    return f(dst2d)


def _xw_body(y_ref, r_ref, mems_ref, w_ref, h_ref, u_ref, dist_ref, dv_ref):
    i = pl.program_id(0)
    y = y_ref[0]
    rows = mems_ref[...]
    ridx = jax.lax.broadcasted_iota(jnp.int32, (BLK, 1), 0) + i * BLK
    rvec = r_ref[...]
    sel = jnp.broadcast_to(ridx == y, (BLK, Z))
    rows = jnp.where(sel, jnp.broadcast_to(rvec, (BLK, Z)), rows)
    gc = jax.lax.broadcasted_iota(jnp.int32, (1, BLK), 1) + i * BLK
    hblk = h_ref[...]
    deg = (hblk[0, 0] + hblk[0, 1] + 2.0).reshape(1, BLK)
    deg = jnp.where(gc == y, float(KNN + 2), deg)
    dinv = jax.lax.rsqrt(deg)
    dv_ref[...] = dinv.reshape(1, 1, BLK)
    dcol = dinv.reshape(BLK, 1)
    rows_ok = jnp.broadcast_to(ridx < M, (BLK, Z))
    xw = jnp.dot(rows, w_ref[...], preferred_element_type=jnp.float32)
    u_ref[...] = jnp.where(rows_ok, dcol * xw, 0.0)
    mr = lax.dot_general(rvec, rows, (((1,), (1,)), ((), ())),
                         preferred_element_type=jnp.float32)
    sq = rows * rows
    ssq = lax.dot_general(jnp.ones((1, Z), jnp.float32), sq,
                          (((1,), (1,)), ((), ())),
                          preferred_element_type=jnp.float32)
    rss = jnp.sum(rvec * rvec)
    dist = ssq - 2.0 * mr + rss
    dist = jnp.where(gc < M, dist, BIGF)
    dist_ref[...] = dist.reshape(1, 1, BLK)


def _xw_call(y, r, mems, W, h3):
    u, dist3, dv3 = pl.pallas_call(
        _xw_body,
        grid=(GRID,),
        in_specs=[
            pl.BlockSpec(memory_space=pltpu.SMEM),
            pl.BlockSpec((1, Z), lambda i: (0, 0)),
            pl.BlockSpec((BLK, Z), lambda i: (i, 0)),
            pl.BlockSpec((Z, Z), lambda i: (0, 0)),
            pl.BlockSpec((1, 2, BLK), lambda i: (i, 0, 0)),
        ],
        out_specs=[
            pl.BlockSpec((BLK, Z), lambda i: (i, 0)),
            pl.BlockSpec((1, 1, BLK), lambda i: (i, 0, 0)),
            pl.BlockSpec((1, 1, BLK), lambda i: (i, 0, 0)),
        ],
        out_shape=[
            jax.ShapeDtypeStruct((MP, Z), jnp.float32),
            jax.ShapeDtypeStruct((GRID, 1, BLK), jnp.float32),
            jax.ShapeDtypeStruct((GRID, 1, BLK), jnp.float32),
        ],
    )(y.astype(jnp.int32), r, mems, W, h3)
    return u, dist3, dv3


def _topk_body(dist_ref, out_ref, xs_ref):
    xs_ref[...] = dist_ref[...].reshape(GRID, BLK)
    fi = (jax.lax.broadcasted_iota(jnp.int32, (GRID, BLK), 0) * BLK
          + jax.lax.broadcasted_iota(jnp.int32, (GRID, BLK), 1))
    for k in range(KNN):
        x = xs_ref[...]
        m = jnp.min(x)
        idx = jnp.min(jnp.where(x == m, fi, BIGI))
        out_ref[k] = idx
        xs_ref[...] = jnp.where(fi == idx, BIGF, x)
    out_ref[7] = 0


def _topk_call(dist3):
    nn = pl.pallas_call(
        _topk_body,
        in_specs=[pl.BlockSpec((GRID, 1, BLK), lambda: (0, 0, 0))],
        out_specs=pl.BlockSpec(memory_space=pltpu.SMEM),
        out_shape=jax.ShapeDtypeStruct((8,), jnp.int32),
        scratch_shapes=[pltpu.VMEM((GRID, BLK), jnp.float32)],
    )(dist3)
    return nn


NWIN = 16
WIN = EPT // NWIN
NVREG = WIN // 16
BCAP = 3072
SRCMASK = (1 << 17) - 1


def _seg_body(u_ref, src_ref, dst_ref, s_ref,
              wsrc, wdst, bucket,
              ss0, sd0, ss1, sd1, ss2, sd2, ss3, sd3,
              rowbuf0, rowbuf1, zrow, acc,
              semz, semg0, semg1, sema0, sema1):
    c = lax.axis_index("c")
    s = lax.axis_index("s")
    lane = lax.iota(jnp.int32, 16)
    z16f = jnp.zeros((16,), jnp.float32)
    cbase = c * (NCH // 2)

    def zb(i, _):
        zrow[i >> 3, pl.ds((i & 7) * 16, 16)] = z16f
        return 0

    lax.fori_loop(0, 32 * 8, zb, 0)

    cnts = tuple(jnp.zeros((16,), jnp.int32) for _ in range(NCH // 2))
    for wi in range(NWIN):
        off = s * EPT + wi * WIN
        pltpu.sync_copy(src_ref.at[pl.ds(off, WIN)], wsrc)
        pltpu.sync_copy(dst_ref.at[pl.ds(off, WIN)], wdst)

        def win_body(j, cnts):
            d = wdst[pl.ds(j * 16, 16)]
            sv = wsrc[pl.ds(j * 16, 16)]
            chrel = (d >> 13) - cbase
            val = sv | ((d & (R - 1)) << 17)
            new = []
            for pb in range(NCH // 2):
                m = chrel == pb
                pos = cnts[pb] + plsc.cumsum(m.astype(jnp.int32)) - 1
                plsc.store_scatter(bucket, [pb * BCAP + pos], val, mask=m)
                new.append(cnts[pb] + plsc.all_reduce_population_count(m))
            return tuple(new)

        cnts = lax.fori_loop(0, NVREG, win_body, cnts)

    def unpack(goff, pb, stage_s, stage_d):
        for tt in range(8):
            v = bucket[pl.ds(pb * BCAP + goff * 128 + tt * 16, 16)]
            stage_s[pl.ds(tt * 16, 16)] = v & SRCMASK
            stage_d[0, pl.ds(tt * 16, 16)] = v >> 17

    for pb in range(NCH // 2):
        base = (cbase + pb) * R
        zd = [pltpu.async_copy(zrow, acc.at[pl.ds(s * OUT_PT + z * 32, 32)],
                               semz) for z in range(16)]

        cnt = cnts[pb]
        cnt_s = jnp.max(cnt)
        ncap_s = (cnt_s + 255) & (-256)
        for tt in range(16):
            posv = cnt + tt * 16 + lane
            mm = posv < ncap_s
            padval = M + (posv & 255)
            plsc.store_scatter(bucket, [pb * BCAP + posv], padval, mask=mm)

        for d in zd:
            d.wait()
        plsc.subcore_barrier()

        ngrp = ncap_s >> 7
        nq = ngrp >> 2

        def quad(i, _):
            g = i * 4
            unpack(g, pb, ss0, sd0)
            dA = pltpu.async_copy(u_ref.at[ss0], rowbuf0, semg0)
            unpack(g + 1, pb, ss1, sd1)
            dB = pltpu.async_copy(u_ref.at[ss1], rowbuf1, semg1)
            dA.wait()
            aA = pltpu.async_copy(rowbuf0, acc.at[sd0.at[0]], sema0,
                                  add=True)
            dB.wait()
            aB = pltpu.async_copy(rowbuf1, acc.at[sd1.at[0]], sema1,
                                  add=True)
            unpack(g + 2, pb, ss2, sd2)
            aA.wait()
            dC = pltpu.async_copy(u_ref.at[ss2], rowbuf0, semg0)
            unpack(g + 3, pb, ss3, sd3)
            aB.wait()
            dD = pltpu.async_copy(u_ref.at[ss3], rowbuf1, semg1)
            dC.wait()
            aC = pltpu.async_copy(rowbuf0, acc.at[sd2.at[0]], sema0,
                                  add=True)
            dD.wait()
            aD = pltpu.async_copy(rowbuf1, acc.at[sd3.at[0]], sema1,
                                  add=True)
            aC.wait()
            aD.wait()
            return 0

        lax.fori_loop(0, nq, quad, 0)

        @pl.when((ngrp & 3) == 2)
        def _():
            g = nq * 4
            unpack(g, pb, ss0, sd0)
            dA = pltpu.async_copy(u_ref.at[ss0], rowbuf0, semg0)
            unpack(g + 1, pb, ss1, sd1)
            dB = pltpu.async_copy(u_ref.at[ss1], rowbuf1, semg1)
            dA.wait()
            aA = pltpu.async_copy(rowbuf0, acc.at[sd0.at[0]], sema0,
                                  add=True)
            dB.wait()
            aB = pltpu.async_copy(rowbuf1, acc.at[sd1.at[0]], sema1,
                                  add=True)
            aA.wait()
            aB.wait()

        plsc.subcore_barrier()
        pltpu.sync_copy(acc.at[pl.ds(s * OUT_PT, OUT_PT)],
                        s_ref.at[pl.ds(base + s * OUT_PT, OUT_PT)])
        plsc.subcore_barrier()


def _seg_call(u, src1d, dst1d):
    f = pl.kernel(
        _seg_body,
        out_type=jax.ShapeDtypeStruct((SP, Z), jnp.float32),
        mesh=_get_mesh(),
        compiler_params=pltpu.CompilerParams(needs_layout_passes=False),
        scratch_types=[
            pltpu.VMEM((WIN,), jnp.int32),
            pltpu.VMEM((WIN,), jnp.int32),
            pltpu.VMEM((7 * BCAP,), jnp.int32),
            pltpu.VMEM((128,), jnp.int32),
            pltpu.VMEM((1, 128), jnp.int32),
            pltpu.VMEM((128,), jnp.int32),
            pltpu.VMEM((1, 128), jnp.int32),
            pltpu.VMEM((128,), jnp.int32),
            pltpu.VMEM((1, 128), jnp.int32),
            pltpu.VMEM((128,), jnp.int32),
            pltpu.VMEM((1, 128), jnp.int32),
            pltpu.VMEM((128, Z), jnp.float32),
            pltpu.VMEM((128, Z), jnp.float32),
            pltpu.VMEM((32, 128), jnp.float32),
            pltpu.VMEM_SHARED((R, Z), jnp.float32),
            pltpu.SemaphoreType.DMA,
            pltpu.SemaphoreType.DMA,
            pltpu.SemaphoreType.DMA,
            pltpu.SemaphoreType.DMA,
            pltpu.SemaphoreType.DMA,
        ],
    )
    return f(u, src1d, dst1d)


def kernel(y, r, mems, edge_index, W):
    y0 = y[0].astype(jnp.int32)
    E = edge_index.shape[1]
    src_e = edge_index[0].astype(jnp.int32)
    dst_e = edge_index[1].astype(jnp.int32)
    pad_n = E_PAD - E - KNN

    spread_e = MP + (jnp.arange(E, dtype=jnp.int32) % (SP - MP))
    dst_m = jnp.where(dst_e == y0, spread_e, dst_e)
    pad_dst = MP + (jnp.arange(pad_n, dtype=jnp.int32) % (SP - MP))
    pad_src = M + (jnp.arange(pad_n, dtype=jnp.int32) % (MP - M))

    pad_n_h = E_PAD_H - E
    pad_dst_h = MP + (jnp.arange(pad_n_h, dtype=jnp.int32) % (SP - MP))
    dst_hist = jnp.concatenate([dst_m, pad_dst_h])
    h = _hist_call(dst_hist.reshape(EROWS_H, 128))
    h3 = h.reshape(2, SP // BLK, BLK).transpose(1, 0, 2)

    u, dist3, dv3 = _xw_call(y, r, mems, W, h3)
    nn8 = _topk_call(dist3)
    nn_idx = nn8[:KNN]

    src_all = jnp.concatenate([src_e, nn_idx, pad_src])
    dst_all = jnp.concatenate(
        [dst_m, jnp.full((KNN,), y0, jnp.int32), pad_dst])
    s = _seg_call(u, src_all, dst_all)

    dv = dv3.reshape(MP)[:M]
    out = dv[:, None] * (s[:M] + 2.0 * u[:M])
    return out


# speedup vs baseline: 40.4153x; 1.0029x over previous
import jax
import jax.numpy as jnp
import numpy as np
from jax import lax
from jax.experimental import pallas as pl
from jax.experimental.pallas import tpu as pltpu
from jax.experimental.pallas import tpu_sc as plsc

M = 100000
Z = 128
KNN = 7
BLK = 1024
GRID = 98
MP = GRID * BLK
R = 8192
NCH = 14
SP = NCH * R
E_PAD = 503808
EPT = E_PAD // 16
EROWS_H = 4096
E_PAD_H = EROWS_H * 128
HROWS = EROWS_H // 32
HIST_PT = SP // 16
CAP = 5120
ACC_PT = 520
OUT_PT = R // 16
BIGF = 3e38
BIGI = 2 ** 30

_mesh_cache = []


def _get_mesh():
    if not _mesh_cache:
        _mesh_cache.append(plsc.VectorSubcoreMesh(
            core_axis_name="c", subcore_axis_name="s"))
    return _mesh_cache[0]


def _hist_body(dst_ref, h_ref, dst_v, ones_v, zbuf, h_sp):
    c = lax.axis_index("c")
    s = lax.axis_index("s")
    w = c * 16 + s

    def zb(i, _):
        zbuf[pl.ds(i * 16, 16)] = jnp.zeros((16,), jnp.float32)
        return 0

    lax.fori_loop(0, HIST_PT // 16, zb, 0)

    def ob(i, _):
        ones_v[pl.ds(i * 16, 16)] = jnp.ones((16,), jnp.float32)
        return 0

    lax.fori_loop(0, 8, ob, 0)

    pltpu.sync_copy(dst_ref.at[pl.ds(w * HROWS, HROWS)], dst_v)
    pltpu.sync_copy(zbuf, h_sp.at[pl.ds(s * HIST_PT, HIST_PT)])
    plsc.subcore_barrier()

    def add(j, _):
        pltpu.sync_copy(ones_v, h_sp.at[dst_v.at[j]], add=True)
        return 0

    lax.fori_loop(0, HROWS, add, 0)
    plsc.subcore_barrier()
    pltpu.sync_copy(h_sp.at[pl.ds(s * HIST_PT, HIST_PT)],
                    h_ref.at[c, pl.ds(s * HIST_PT, HIST_PT)])


def _hist_call(dst2d):
    f = pl.kernel(
        _hist_body,
        out_type=jax.ShapeDtypeStruct((2, SP), jnp.float32),
        mesh=_get_mesh(),
        compiler_params=pltpu.CompilerParams(needs_layout_passes=False),
        scratch_types=[
            pltpu.VMEM((HROWS, 128), jnp.int32),
            pltpu.VMEM((128,), jnp.float32),
            pltpu.VMEM((HIST_PT,), jnp.float32),
            pltpu.VMEM_SHARED((SP,), jnp.float32),
        ],
    )
    return f(dst2d)


def _xw_body(y_ref, r_ref, mems_ref, w_ref, h_ref, u_ref, dist_ref, dv_ref):
    i = pl.program_id(0)
    y = y_ref[0]
    rows = mems_ref[...]
    ridx = jax.lax.broadcasted_iota(jnp.int32, (BLK, 1), 0) + i * BLK
    rvec = r_ref[...]
    sel = jnp.broadcast_to(ridx == y, (BLK, Z))
    rows = jnp.where(sel, jnp.broadcast_to(rvec, (BLK, Z)), rows)
    gc = jax.lax.broadcasted_iota(jnp.int32, (1, BLK), 1) + i * BLK
    hblk = h_ref[...]
    deg = (hblk[0, 0] + hblk[0, 1] + 2.0).reshape(1, BLK)
    deg = jnp.where(gc == y, float(KNN + 2), deg)
    dinv = jax.lax.rsqrt(deg)
    dv_ref[...] = dinv.reshape(1, 1, BLK)
    dcol = dinv.reshape(BLK, 1)
    rows_ok = jnp.broadcast_to(ridx < M, (BLK, Z))
    xw = jnp.dot(rows, w_ref[...], preferred_element_type=jnp.float32)
    u_ref[...] = jnp.where(rows_ok, dcol * xw, 0.0)
    mr = lax.dot_general(rvec, rows, (((1,), (1,)), ((), ())),
                         preferred_element_type=jnp.float32)
    sq = rows * rows
    ssq = lax.dot_general(jnp.ones((1, Z), jnp.float32), sq,
                          (((1,), (1,)), ((), ())),
                          preferred_element_type=jnp.float32)
    rss = jnp.sum(rvec * rvec)
    dist = ssq - 2.0 * mr + rss
    dist = jnp.where(gc < M, dist, BIGF)
    dist_ref[...] = dist.reshape(1, 1, BLK)


def _xw_call(y, r, mems, W, h3):
    u, dist3, dv3 = pl.pallas_call(
        _xw_body,
        grid=(GRID,),
        in_specs=[
            pl.BlockSpec(memory_space=pltpu.SMEM),
            pl.BlockSpec((1, Z), lambda i: (0, 0)),
            pl.BlockSpec((BLK, Z), lambda i: (i, 0)),
            pl.BlockSpec((Z, Z), lambda i: (0, 0)),
            pl.BlockSpec((1, 2, BLK), lambda i: (i, 0, 0)),
        ],
        out_specs=[
            pl.BlockSpec((BLK, Z), lambda i: (i, 0)),
            pl.BlockSpec((1, 1, BLK), lambda i: (i, 0, 0)),
            pl.BlockSpec((1, 1, BLK), lambda i: (i, 0, 0)),
        ],
        out_shape=[
            jax.ShapeDtypeStruct((MP, Z), jnp.float32),
            jax.ShapeDtypeStruct((GRID, 1, BLK), jnp.float32),
            jax.ShapeDtypeStruct((GRID, 1, BLK), jnp.float32),
        ],
    )(y.astype(jnp.int32), r, mems, W, h3)
    return u, dist3, dv3


def _topk_body(dist_ref, out_ref, xs_ref):
    xs_ref[...] = dist_ref[...].reshape(GRID, BLK)
    fi = (jax.lax.broadcasted_iota(jnp.int32, (GRID, BLK), 0) * BLK
          + jax.lax.broadcasted_iota(jnp.int32, (GRID, BLK), 1))
    for k in range(KNN):
        x = xs_ref[...]
        m = jnp.min(x)
        idx = jnp.min(jnp.where(x == m, fi, BIGI))
        out_ref[k] = idx
        xs_ref[...] = jnp.where(fi == idx, BIGF, x)
    out_ref[7] = 0


def _topk_call(dist3):
    nn = pl.pallas_call(
        _topk_body,
        in_specs=[pl.BlockSpec((GRID, 1, BLK), lambda: (0, 0, 0))],
        out_specs=pl.BlockSpec(memory_space=pltpu.SMEM),
        out_shape=jax.ShapeDtypeStruct((8,), jnp.int32),
        scratch_shapes=[pltpu.VMEM((GRID, BLK), jnp.float32)],
    )(dist3)
    return nn


NWIN = 16
WIN = EPT // NWIN
NVREG = WIN // 16
BCAP = 3072
SRCMASK = (1 << 17) - 1


def _seg_body(u_ref, src_ref, dst_ref, s_ref,
              wsrc, wdst, bucket,
              ss0, sd0, ss1, sd1, ss2, sd2, ss3, sd3,
              rowbuf0, rowbuf1, zrow, acc,
              semz, semg0, semg1, sema0, sema1):
    c = lax.axis_index("c")
    s = lax.axis_index("s")
    lane = lax.iota(jnp.int32, 16)
    z16f = jnp.zeros((16,), jnp.float32)
    cbase = c * (NCH // 2)

    def zb(i, _):
        zrow[i >> 3, pl.ds((i & 7) * 16, 16)] = z16f
        return 0

    lax.fori_loop(0, 32 * 8, zb, 0)

    cnts = tuple(jnp.zeros((16,), jnp.int32) for _ in range(NCH // 2))
    for wi in range(NWIN):
        off = s * EPT + wi * WIN
        pltpu.sync_copy(src_ref.at[pl.ds(off, WIN)], wsrc)
        pltpu.sync_copy(dst_ref.at[pl.ds(off, WIN)], wdst)

        def win_body(j, cnts):
            d = wdst[pl.ds(j * 16, 16)]
            sv = wsrc[pl.ds(j * 16, 16)]
            chrel = (d >> 13) - cbase
            val = sv | ((d & (R - 1)) << 17)
            new = []
            for pb in range(NCH // 2):
                m = chrel == pb
                pos = cnts[pb] + plsc.cumsum(m.astype(jnp.int32)) - 1
                plsc.store_scatter(bucket, [pb * BCAP + pos], val, mask=m)
                new.append(cnts[pb] + plsc.all_reduce_population_count(m))
            return tuple(new)

        cnts = lax.fori_loop(0, NVREG, win_body, cnts)

    def unpack(goff, pb, stage_s, stage_d):
        for tt in range(8):
            v = bucket[pl.ds(pb * BCAP + goff * 128 + tt * 16, 16)]
            stage_s[pl.ds(tt * 16, 16)] = v & SRCMASK
            stage_d[0, pl.ds(tt * 16, 16)] = v >> 17

    for pb in range(NCH // 2):
        base = (cbase + pb) * R
        zd = [pltpu.async_copy(zrow, acc.at[pl.ds(s * OUT_PT + z * 32, 32)],
                               semz) for z in range(16)]

        cnt = cnts[pb]
        cnt_s = jnp.max(cnt)
        ncap_s = (cnt_s + 255) & (-256)
        for tt in range(16):
            posv = cnt + tt * 16 + lane
            mm = posv < ncap_s
            padval = M + (posv & 255)
            plsc.store_scatter(bucket, [pb * BCAP + posv], padval, mask=mm)

        for d in zd:
            d.wait()
        plsc.subcore_barrier()

        ngrp = ncap_s >> 7
        nq = ngrp >> 2

        def quad(i, _):
            g = i * 4
            unpack(g, pb, ss0, sd0)
            dA = pltpu.async_copy(u_ref.at[ss0], rowbuf0, semg0)
            unpack(g + 1, pb, ss1, sd1)
            dB = pltpu.async_copy(u_ref.at[ss1], rowbuf1, semg1)
            dA.wait()
            aA = pltpu.async_copy(rowbuf0, acc.at[sd0.at[0]], sema0,
                                  add=True)
            dB.wait()
            aB = pltpu.async_copy(rowbuf1, acc.at[sd1.at[0]], sema1,
                                  add=True)
            unpack(g + 2, pb, ss2, sd2)
            aA.wait()
            dC = pltpu.async_copy(u_ref.at[ss2], rowbuf0, semg0)
            unpack(g + 3, pb, ss3, sd3)
            aB.wait()
            dD = pltpu.async_copy(u_ref.at[ss3], rowbuf1, semg1)
            dC.wait()
            aC = pltpu.async_copy(rowbuf0, acc.at[sd2.at[0]], sema0,
                                  add=True)
            dD.wait()
            aD = pltpu.async_copy(rowbuf1, acc.at[sd3.at[0]], sema1,
                                  add=True)
            aC.wait()
            aD.wait()
            return 0

        lax.fori_loop(0, nq, quad, 0)

        @pl.when((ngrp & 3) == 2)
        def _():
            g = nq * 4
            unpack(g, pb, ss0, sd0)
            dA = pltpu.async_copy(u_ref.at[ss0], rowbuf0, semg0)
            unpack(g + 1, pb, ss1, sd1)
            dB = pltpu.async_copy(u_ref.at[ss1], rowbuf1, semg1)
            dA.wait()
            aA = pltpu.async_copy(rowbuf0, acc.at[sd0.at[0]], sema0,
                                  add=True)
            dB.wait()
            aB = pltpu.async_copy(rowbuf1, acc.at[sd1.at[0]], sema1,
                                  add=True)
            aA.wait()
            aB.wait()

        plsc.subcore_barrier()
        pltpu.sync_copy(acc.at[pl.ds(s * OUT_PT, OUT_PT)],
                        s_ref.at[pl.ds(base + s * OUT_PT, OUT_PT)])
        plsc.subcore_barrier()


def _seg_call(u, src1d, dst1d):
    f = pl.kernel(
        _seg_body,
        out_type=jax.ShapeDtypeStruct((SP, Z), jnp.float32),
        mesh=_get_mesh(),
        compiler_params=pltpu.CompilerParams(needs_layout_passes=False),
        scratch_types=[
            pltpu.VMEM((WIN,), jnp.int32),
            pltpu.VMEM((WIN,), jnp.int32),
            pltpu.VMEM((7 * BCAP,), jnp.int32),
            pltpu.VMEM((128,), jnp.int32),
            pltpu.VMEM((1, 128), jnp.int32),
            pltpu.VMEM((128,), jnp.int32),
            pltpu.VMEM((1, 128), jnp.int32),
            pltpu.VMEM((128,), jnp.int32),
            pltpu.VMEM((1, 128), jnp.int32),
            pltpu.VMEM((128,), jnp.int32),
            pltpu.VMEM((1, 128), jnp.int32),
            pltpu.VMEM((128, Z), jnp.float32),
            pltpu.VMEM((128, Z), jnp.float32),
            pltpu.VMEM((32, 128), jnp.float32),
            pltpu.VMEM_SHARED((R, Z), jnp.float32),
            pltpu.SemaphoreType.DMA,
            pltpu.SemaphoreType.DMA,
            pltpu.SemaphoreType.DMA,
            pltpu.SemaphoreType.DMA,
            pltpu.SemaphoreType.DMA,
        ],
    )
    return f(u, src1d, dst1d)


_E_FIXED = 500000
_SPREAD_E = np.int32(MP) + (np.arange(_E_FIXED, dtype=np.int32) % (SP - MP))
_PAD_DST = np.int32(MP) + (
    np.arange(E_PAD - _E_FIXED - KNN, dtype=np.int32) % (SP - MP))
_PAD_SRC = np.int32(M) + (
    np.arange(E_PAD - _E_FIXED - KNN, dtype=np.int32) % (MP - M))
_PAD_DST_H = np.int32(MP) + (
    np.arange(E_PAD_H - _E_FIXED, dtype=np.int32) % (SP - MP))


def kernel(y, r, mems, edge_index, W):
    y0 = y[0].astype(jnp.int32)
    src_e = edge_index[0].astype(jnp.int32)
    dst_e = edge_index[1].astype(jnp.int32)

    dst_m = jnp.where(dst_e == y0, jnp.asarray(_SPREAD_E), dst_e)
    pad_dst = jnp.asarray(_PAD_DST)
    pad_src = jnp.asarray(_PAD_SRC)

    dst_hist = jnp.concatenate([dst_m, jnp.asarray(_PAD_DST_H)])
    h = _hist_call(dst_hist.reshape(EROWS_H, 128))
    h3 = h.reshape(2, SP // BLK, BLK).transpose(1, 0, 2)

    u, dist3, dv3 = _xw_call(y, r, mems, W, h3)
    nn8 = _topk_call(dist3)
    nn_idx = nn8[:KNN]

    src_all = jnp.concatenate([src_e, nn_idx, pad_src])
    dst_all = jnp.concatenate(
        [dst_m, jnp.full((KNN,), y0, jnp.int32), pad_dst])
    s = _seg_call(u, src_all, dst_all)

    dv = dv3.reshape(MP)[:M]
    out = dv[:, None] * (s[:M] + 2.0 * u[:M])
    return out


# speedup vs baseline: 41.9562x; 1.0381x over previous
import jax
import jax.numpy as jnp
import numpy as np
from jax import lax
from jax.experimental import pallas as pl
from jax.experimental.pallas import tpu as pltpu
from jax.experimental.pallas import tpu_sc as plsc

M = 100000
Z = 128
KNN = 7
BLK = 2048
GRID = 49
MP = GRID * BLK
R = 8192
NCH = 14
SP = NCH * R
E_PAD = 503808
EPT = E_PAD // 16
EROWS_H = 4096
E_PAD_H = EROWS_H * 128
HROWS = EROWS_H // 32
HIST_PT = SP // 16
CAP = 5120
ACC_PT = 520
OUT_PT = R // 16
BIGF = 3e38
BIGI = 2 ** 30

_mesh_cache = []


def _get_mesh():
    if not _mesh_cache:
        _mesh_cache.append(plsc.VectorSubcoreMesh(
            core_axis_name="c", subcore_axis_name="s"))
    return _mesh_cache[0]


def _hist_body(dst_ref, h_ref, dst_v, ones_v, zbuf, h_sp):
    c = lax.axis_index("c")
    s = lax.axis_index("s")
    w = c * 16 + s

    def zb(i, _):
        zbuf[pl.ds(i * 16, 16)] = jnp.zeros((16,), jnp.float32)
        return 0

    lax.fori_loop(0, HIST_PT // 16, zb, 0)

    def ob(i, _):
        ones_v[pl.ds(i * 16, 16)] = jnp.ones((16,), jnp.float32)
        return 0

    lax.fori_loop(0, 8, ob, 0)

    pltpu.sync_copy(dst_ref.at[pl.ds(w * HROWS, HROWS)], dst_v)
    pltpu.sync_copy(zbuf, h_sp.at[pl.ds(s * HIST_PT, HIST_PT)])
    plsc.subcore_barrier()

    def add(j, _):
        pltpu.sync_copy(ones_v, h_sp.at[dst_v.at[j]], add=True)
        return 0

    lax.fori_loop(0, HROWS, add, 0)
    plsc.subcore_barrier()
    pltpu.sync_copy(h_sp.at[pl.ds(s * HIST_PT, HIST_PT)],
                    h_ref.at[c, pl.ds(s * HIST_PT, HIST_PT)])


def _hist_call(dst2d):
    f = pl.kernel(
        _hist_body,
        out_type=jax.ShapeDtypeStruct((2, SP), jnp.float32),
        mesh=_get_mesh(),
        compiler_params=pltpu.CompilerParams(needs_layout_passes=False),
        scratch_types=[
            pltpu.VMEM((HROWS, 128), jnp.int32),
            pltpu.VMEM((128,), jnp.float32),
            pltpu.VMEM((HIST_PT,), jnp.float32),
            pltpu.VMEM_SHARED((SP,), jnp.float32),
        ],
    )
    return f(dst2d)


def _xw_body(y_ref, r_ref, mems_ref, w_ref, h_ref, u_ref, dist_ref, dv_ref):
    i = pl.program_id(0)
    y = y_ref[0]
    rows = mems_ref[...]
    ridx = jax.lax.broadcasted_iota(jnp.int32, (BLK, 1), 0) + i * BLK
    rvec = r_ref[...]
    sel = jnp.broadcast_to(ridx == y, (BLK, Z))
    rows = jnp.where(sel, jnp.broadcast_to(rvec, (BLK, Z)), rows)
    gc = jax.lax.broadcasted_iota(jnp.int32, (1, BLK), 1) + i * BLK
    hblk = h_ref[...]
    deg = (hblk[0, 0] + hblk[0, 1] + 2.0).reshape(1, BLK)
    deg = jnp.where(gc == y, float(KNN + 2), deg)
    dinv = jax.lax.rsqrt(deg)
    dv_ref[...] = dinv.reshape(1, 1, BLK)
    dcol = dinv.reshape(BLK, 1)
    rows_ok = jnp.broadcast_to(ridx < M, (BLK, Z))
    xw = jnp.dot(rows, w_ref[...], preferred_element_type=jnp.float32)
    u_ref[...] = jnp.where(rows_ok, dcol * xw, 0.0)
    mr = lax.dot_general(rvec, rows, (((1,), (1,)), ((), ())),
                         preferred_element_type=jnp.float32)
    sq = rows * rows
    ssq = lax.dot_general(jnp.ones((1, Z), jnp.float32), sq,
                          (((1,), (1,)), ((), ())),
                          preferred_element_type=jnp.float32)
    rss = jnp.sum(rvec * rvec)
    dist = ssq - 2.0 * mr + rss
    dist = jnp.where(gc < M, dist, BIGF)
    dist_ref[...] = dist.reshape(1, 1, BLK)


def _xw_call(y, r, mems, W, h3):
    u, dist3, dv3 = pl.pallas_call(
        _xw_body,
        grid=(GRID,),
        in_specs=[
            pl.BlockSpec(memory_space=pltpu.SMEM),
            pl.BlockSpec((1, Z), lambda i: (0, 0)),
            pl.BlockSpec((BLK, Z), lambda i: (i, 0)),
            pl.BlockSpec((Z, Z), lambda i: (0, 0)),
            pl.BlockSpec((1, 2, BLK), lambda i: (i, 0, 0)),
        ],
        out_specs=[
            pl.BlockSpec((BLK, Z), lambda i: (i, 0)),
            pl.BlockSpec((1, 1, BLK), lambda i: (i, 0, 0)),
            pl.BlockSpec((1, 1, BLK), lambda i: (i, 0, 0)),
        ],
        out_shape=[
            jax.ShapeDtypeStruct((MP, Z), jnp.float32),
            jax.ShapeDtypeStruct((GRID, 1, BLK), jnp.float32),
            jax.ShapeDtypeStruct((GRID, 1, BLK), jnp.float32),
        ],
    )(y.astype(jnp.int32), r, mems, W, h3)
    return u, dist3, dv3


def _topk_body(dist_ref, out_ref, xs_ref):
    xs_ref[...] = dist_ref[...].reshape(GRID, BLK)
    fi = (jax.lax.broadcasted_iota(jnp.int32, (GRID, BLK), 0) * BLK
          + jax.lax.broadcasted_iota(jnp.int32, (GRID, BLK), 1))
    for k in range(KNN):
        x = xs_ref[...]
        m = jnp.min(x)
        idx = jnp.min(jnp.where(x == m, fi, BIGI))
        out_ref[k] = idx
        xs_ref[...] = jnp.where(fi == idx, BIGF, x)
    out_ref[7] = 0


def _topk_call(dist3):
    nn = pl.pallas_call(
        _topk_body,
        in_specs=[pl.BlockSpec((GRID, 1, BLK), lambda: (0, 0, 0))],
        out_specs=pl.BlockSpec(memory_space=pltpu.SMEM),
        out_shape=jax.ShapeDtypeStruct((8,), jnp.int32),
        scratch_shapes=[pltpu.VMEM((GRID, BLK), jnp.float32)],
    )(dist3)
    return nn


NWIN = 16
WIN = EPT // NWIN
NVREG = WIN // 16
BCAP = 3072
SRCMASK = (1 << 17) - 1


def _seg_body(u_ref, src_ref, dst_ref, s_ref,
              wsrc, wdst, bucket,
              ss0, sd0, ss1, sd1, ss2, sd2, ss3, sd3,
              rowbuf0, rowbuf1, zrow, acc,
              semz, semg0, semg1, sema0, sema1):
    c = lax.axis_index("c")
    s = lax.axis_index("s")
    lane = lax.iota(jnp.int32, 16)
    z16f = jnp.zeros((16,), jnp.float32)
    cbase = c * (NCH // 2)

    def zb(i, _):
        zrow[i >> 3, pl.ds((i & 7) * 16, 16)] = z16f
        return 0

    lax.fori_loop(0, 32 * 8, zb, 0)

    cnts = tuple(jnp.zeros((16,), jnp.int32) for _ in range(NCH // 2))
    for wi in range(NWIN):
        off = s * EPT + wi * WIN
        pltpu.sync_copy(src_ref.at[pl.ds(off, WIN)], wsrc)
        pltpu.sync_copy(dst_ref.at[pl.ds(off, WIN)], wdst)

        def win_body(j, cnts):
            d = wdst[pl.ds(j * 16, 16)]
            sv = wsrc[pl.ds(j * 16, 16)]
            chrel = (d >> 13) - cbase
            val = sv | ((d & (R - 1)) << 17)
            new = []
            for pb in range(NCH // 2):
                m = chrel == pb
                pos = cnts[pb] + plsc.cumsum(m.astype(jnp.int32)) - 1
                plsc.store_scatter(bucket, [pb * BCAP + pos], val, mask=m)
                new.append(cnts[pb] + plsc.all_reduce_population_count(m))
            return tuple(new)

        cnts = lax.fori_loop(0, NVREG, win_body, cnts)

    def unpack(goff, pb, stage_s, stage_d):
        for tt in range(8):
            v = bucket[pl.ds(pb * BCAP + goff * 128 + tt * 16, 16)]
            stage_s[pl.ds(tt * 16, 16)] = v & SRCMASK
            stage_d[0, pl.ds(tt * 16, 16)] = v >> 17

    for pb in range(NCH // 2):
        base = (cbase + pb) * R
        zd = [pltpu.async_copy(zrow, acc.at[pl.ds(s * OUT_PT + z * 32, 32)],
                               semz) for z in range(16)]

        cnt = cnts[pb]
        cnt_s = jnp.max(cnt)
        ncap_s = (cnt_s + 255) & (-256)
        for tt in range(16):
            posv = cnt + tt * 16 + lane
            mm = posv < ncap_s
            padval = M + (posv & 255)
            plsc.store_scatter(bucket, [pb * BCAP + posv], padval, mask=mm)

        for d in zd:
            d.wait()
        plsc.subcore_barrier()

        ngrp = ncap_s >> 7
        nq = ngrp >> 2

        def quad(i, _):
            g = i * 4
            unpack(g, pb, ss0, sd0)
            dA = pltpu.async_copy(u_ref.at[ss0], rowbuf0, semg0)
            unpack(g + 1, pb, ss1, sd1)
            dB = pltpu.async_copy(u_ref.at[ss1], rowbuf1, semg1)
            dA.wait()
            aA = pltpu.async_copy(rowbuf0, acc.at[sd0.at[0]], sema0,
                                  add=True)
            dB.wait()
            aB = pltpu.async_copy(rowbuf1, acc.at[sd1.at[0]], sema1,
                                  add=True)
            unpack(g + 2, pb, ss2, sd2)
            aA.wait()
            dC = pltpu.async_copy(u_ref.at[ss2], rowbuf0, semg0)
            unpack(g + 3, pb, ss3, sd3)
            aB.wait()
            dD = pltpu.async_copy(u_ref.at[ss3], rowbuf1, semg1)
            dC.wait()
            aC = pltpu.async_copy(rowbuf0, acc.at[sd2.at[0]], sema0,
                                  add=True)
            dD.wait()
            aD = pltpu.async_copy(rowbuf1, acc.at[sd3.at[0]], sema1,
                                  add=True)
            aC.wait()
            aD.wait()
            return 0

        lax.fori_loop(0, nq, quad, 0)

        @pl.when((ngrp & 3) == 2)
        def _():
            g = nq * 4
            unpack(g, pb, ss0, sd0)
            dA = pltpu.async_copy(u_ref.at[ss0], rowbuf0, semg0)
            unpack(g + 1, pb, ss1, sd1)
            dB = pltpu.async_copy(u_ref.at[ss1], rowbuf1, semg1)
            dA.wait()
            aA = pltpu.async_copy(rowbuf0, acc.at[sd0.at[0]], sema0,
                                  add=True)
            dB.wait()
            aB = pltpu.async_copy(rowbuf1, acc.at[sd1.at[0]], sema1,
                                  add=True)
            aA.wait()
            aB.wait()

        plsc.subcore_barrier()
        pltpu.sync_copy(acc.at[pl.ds(s * OUT_PT, OUT_PT)],
                        s_ref.at[pl.ds(base + s * OUT_PT, OUT_PT)])
        plsc.subcore_barrier()


def _seg_call(u, src1d, dst1d):
    f = pl.kernel(
        _seg_body,
        out_type=jax.ShapeDtypeStruct((SP, Z), jnp.float32),
        mesh=_get_mesh(),
        compiler_params=pltpu.CompilerParams(needs_layout_passes=False),
        scratch_types=[
            pltpu.VMEM((WIN,), jnp.int32),
            pltpu.VMEM((WIN,), jnp.int32),
            pltpu.VMEM((7 * BCAP,), jnp.int32),
            pltpu.VMEM((128,), jnp.int32),
            pltpu.VMEM((1, 128), jnp.int32),
            pltpu.VMEM((128,), jnp.int32),
            pltpu.VMEM((1, 128), jnp.int32),
            pltpu.VMEM((128,), jnp.int32),
            pltpu.VMEM((1, 128), jnp.int32),
            pltpu.VMEM((128,), jnp.int32),
            pltpu.VMEM((1, 128), jnp.int32),
            pltpu.VMEM((128, Z), jnp.float32),
            pltpu.VMEM((128, Z), jnp.float32),
            pltpu.VMEM((32, 128), jnp.float32),
            pltpu.VMEM_SHARED((R, Z), jnp.float32),
            pltpu.SemaphoreType.DMA,
            pltpu.SemaphoreType.DMA,
            pltpu.SemaphoreType.DMA,
            pltpu.SemaphoreType.DMA,
            pltpu.SemaphoreType.DMA,
        ],
    )
    return f(u, src1d, dst1d)


_E_FIXED = 500000
_SPREAD_E = np.int32(MP) + (np.arange(_E_FIXED, dtype=np.int32) % (SP - MP))
_PAD_DST = np.int32(MP) + (
    np.arange(E_PAD - _E_FIXED - KNN, dtype=np.int32) % (SP - MP))
_PAD_SRC = np.int32(M) + (
    np.arange(E_PAD - _E_FIXED - KNN, dtype=np.int32) % (MP - M))
_PAD_DST_H = np.int32(MP) + (
    np.arange(E_PAD_H - _E_FIXED, dtype=np.int32) % (SP - MP))


def kernel(y, r, mems, edge_index, W):
    y0 = y[0].astype(jnp.int32)
    src_e = edge_index[0].astype(jnp.int32)
    dst_e = edge_index[1].astype(jnp.int32)

    dst_m = jnp.where(dst_e == y0, jnp.asarray(_SPREAD_E), dst_e)
    pad_dst = jnp.asarray(_PAD_DST)
    pad_src = jnp.asarray(_PAD_SRC)

    dst_hist = jnp.concatenate([dst_m, jnp.asarray(_PAD_DST_H)])
    h = _hist_call(dst_hist.reshape(EROWS_H, 128))
    h3 = h.reshape(2, SP // BLK, BLK).transpose(1, 0, 2)

    u, dist3, dv3 = _xw_call(y, r, mems, W, h3)
    nn8 = _topk_call(dist3)
    nn_idx = nn8[:KNN]

    src_all = jnp.concatenate([src_e, nn_idx, pad_src])
    dst_all = jnp.concatenate(
        [dst_m, jnp.full((KNN,), y0, jnp.int32), pad_dst])
    s = _seg_call(u, src_all, dst_all)

    dv = dv3.reshape(MP)[:M]
    out = dv[:, None] * (s[:M] + 2.0 * u[:M])
    return out
